# baseline (device time: 72573 ns/iter reference)
import jax
import jax.numpy as jnp
from jax import lax
from jax.experimental import pallas as pl
from jax.experimental.pallas import tpu as pltpu


def kernel(O, Wo):
    B, S, H, D = O.shape
    F = H * D
    N = Wo.shape[1]
    S_half = S // 2

    O2 = O.reshape(B, S, F)

    CH = 4
    ROWS = S_half // CH

    def body(o_hbm, w_hbm, out_hbm, o_vmem, w_vmem, wbf, send_buf,
             recv_buf, own, local_sems, send_sems, recv_sems, out_sems):
        my_x = lax.axis_index("x")
        my_y = lax.axis_index("y")
        peer = (my_x, 1 - my_y)

        o_cp = pltpu.make_async_copy(o_hbm, o_vmem, local_sems.at[0])
        o_cp.start()
        w_cp = pltpu.make_async_copy(w_hbm, w_vmem, local_sems.at[1])
        w_cp.start()

        barrier_sem = pltpu.get_barrier_semaphore()
        pl.semaphore_signal(
            barrier_sem, inc=1, device_id=peer,
            device_id_type=pl.DeviceIdType.MESH,
        )
        pl.semaphore_wait(barrier_sem, 1)

        w_cp.wait()
        wbf[...] = w_vmem[...].astype(jnp.bfloat16)
        o_cp.wait()

        my_lo = my_y * S_half
        peer_lo = (1 - my_y) * S_half

        rdmas = []
        for b in range(B):
            for q in range(CH):
                r0 = q * ROWS
                o_b = o_vmem[b, pl.ds(peer_lo + r0, ROWS), :].astype(
                    jnp.bfloat16
                )
                send_buf[b, r0:r0 + ROWS, :] = jnp.dot(
                    o_b, wbf[...], preferred_element_type=jnp.float32
                ).astype(jnp.bfloat16)
                idx = b * CH + q
                rdma = pltpu.make_async_remote_copy(
                    src_ref=send_buf.at[b, pl.ds(r0, ROWS), :],
                    dst_ref=recv_buf.at[b, pl.ds(r0, ROWS), :],
                    send_sem=send_sems.at[idx],
                    recv_sem=recv_sems.at[idx],
                    device_id=peer,
                    device_id_type=pl.DeviceIdType.MESH,
                )
                rdma.start()
                rdmas.append(rdma)

        for b in range(B):
            o_b = o_vmem[b, pl.ds(my_lo, S_half), :].astype(jnp.bfloat16)
            own[b, :, :] = jnp.dot(
                o_b, wbf[...], preferred_element_type=jnp.float32
            )

        out_cps = []
        for b in range(B):
            for q in range(CH):
                r0 = q * ROWS
                idx = b * CH + q
                rdmas[idx].wait()
                own[b, r0:r0 + ROWS, :] = (
                    own[b, r0:r0 + ROWS, :]
                    + recv_buf[b, r0:r0 + ROWS, :].astype(jnp.float32)
                )
                out_cp = pltpu.make_async_copy(
                    own.at[b, pl.ds(r0, ROWS), :],
                    out_hbm.at[b, pl.ds(r0, ROWS), :],
                    out_sems.at[idx],
                )
                out_cp.start()
                out_cps.append(out_cp)
        for cp in out_cps:
            cp.wait()

    return pl.pallas_call(
        body,
        out_shape=jax.ShapeDtypeStruct((B, S_half, N), jnp.float32),
        in_specs=[
            pl.BlockSpec(memory_space=pl.ANY),
            pl.BlockSpec(memory_space=pl.ANY),
        ],
        out_specs=pl.BlockSpec(memory_space=pl.ANY),
        scratch_shapes=[
            pltpu.VMEM((B, S, F), jnp.float32),
            pltpu.VMEM((F, N), jnp.float32),
            pltpu.VMEM((F, N), jnp.bfloat16),
            pltpu.VMEM((B, S_half, N), jnp.bfloat16),
            pltpu.VMEM((B, S_half, N), jnp.bfloat16),
            pltpu.VMEM((B, S_half, N), jnp.float32),
            pltpu.SemaphoreType.DMA((2,)),
            pltpu.SemaphoreType.DMA((B * CH,)),
            pltpu.SemaphoreType.DMA((B * CH,)),
            pltpu.SemaphoreType.DMA((B * CH,)),
        ],
        compiler_params=pltpu.CompilerParams(
            collective_id=0,
            vmem_limit_bytes=60 * 1024 * 1024,
        ),
    )(O2, Wo)


# device time: 72531 ns/iter; 1.0006x vs baseline; 1.0006x over previous
import jax
import jax.numpy as jnp
from jax import lax
from jax.experimental import pallas as pl
from jax.experimental.pallas import tpu as pltpu


def kernel(O, Wo):
    B, S, H, D = O.shape
    F = H * D
    N = Wo.shape[1]
    S_half = S // 2

    O2 = O.reshape(B, S, F)

    CH = 4
    ROWS = S_half // CH

    def body(o_hbm, w_hbm, out_hbm, o_vmem, w_vmem, wbf, send_buf,
             recv_buf, own, local_sems, send_sems, recv_sems, out_sems):
        my_x = lax.axis_index("x")
        my_y = lax.axis_index("y")
        peer = (my_x, 1 - my_y)

        o_cp = pltpu.make_async_copy(o_hbm, o_vmem, local_sems.at[0])
        o_cp.start()
        w_cp = pltpu.make_async_copy(w_hbm, w_vmem, local_sems.at[1])
        w_cp.start()

        barrier_sem = pltpu.get_barrier_semaphore()
        pl.semaphore_signal(
            barrier_sem, inc=1, device_id=peer,
            device_id_type=pl.DeviceIdType.MESH,
        )
        pl.semaphore_wait(barrier_sem, 1)

        w_cp.wait()
        wbf[...] = w_vmem[...].astype(jnp.bfloat16)
        o_cp.wait()

        my_lo = my_y * S_half
        peer_lo = (1 - my_y) * S_half

        rdmas = []
        for b in range(B):
            for q in range(CH):
                r0 = q * ROWS
                o_b = o_vmem[b, pl.ds(peer_lo + r0, ROWS), :].astype(
                    jnp.bfloat16
                )
                send_buf[b, r0:r0 + ROWS, :] = jnp.dot(
                    o_b, wbf[...], preferred_element_type=jnp.float32
                ).astype(jnp.bfloat16)
                idx = b * CH + q
                rdma = pltpu.make_async_remote_copy(
                    src_ref=send_buf.at[b, pl.ds(r0, ROWS), :],
                    dst_ref=recv_buf.at[b, pl.ds(r0, ROWS), :],
                    send_sem=send_sems.at[idx],
                    recv_sem=recv_sems.at[idx],
                    device_id=peer,
                    device_id_type=pl.DeviceIdType.MESH,
                )
                rdma.start()
                rdmas.append(rdma)

        for b in range(B):
            o_b = o_vmem[b, pl.ds(my_lo, S_half), :].astype(jnp.bfloat16)
            own[b, :, :] = jnp.dot(
                o_b, wbf[...], preferred_element_type=jnp.float32
            )

        out_cps = []
        for b in range(B):
            for q in range(CH):
                r0 = q * ROWS
                idx = b * CH + q
                rdmas[idx].wait()
                own[b, r0:r0 + ROWS, :] = (
                    own[b, r0:r0 + ROWS, :]
                    + recv_buf[b, r0:r0 + ROWS, :].astype(jnp.float32)
                )
                out_cp = pltpu.make_async_copy(
                    own.at[b, pl.ds(r0, ROWS), :],
                    out_hbm.at[b, pl.ds(r0, ROWS), :],
                    out_sems.at[idx],
                )
                out_cp.start()
                out_cps.append(out_cp)
        for cp in out_cps:
            cp.wait()

    return pl.pallas_call(
        body,
        out_shape=jax.ShapeDtypeStruct((B, S_half, N), jnp.float32),
        in_specs=[
            pl.BlockSpec(memory_space=pltpu.MemorySpace.HBM),
            pl.BlockSpec(memory_space=pltpu.MemorySpace.HBM),
        ],
        out_specs=pl.BlockSpec(memory_space=pltpu.MemorySpace.HBM),
        scratch_shapes=[
            pltpu.VMEM((B, S, F), jnp.float32),
            pltpu.VMEM((F, N), jnp.float32),
            pltpu.VMEM((F, N), jnp.bfloat16),
            pltpu.VMEM((B, S_half, N), jnp.bfloat16),
            pltpu.VMEM((B, S_half, N), jnp.bfloat16),
            pltpu.VMEM((B, S_half, N), jnp.float32),
            pltpu.SemaphoreType.DMA((2,)),
            pltpu.SemaphoreType.DMA((B * CH,)),
            pltpu.SemaphoreType.DMA((B * CH,)),
            pltpu.SemaphoreType.DMA((B * CH,)),
        ],
        compiler_params=pltpu.CompilerParams(
            collective_id=0,
            vmem_limit_bytes=60 * 1024 * 1024,
        ),
    )(O2, Wo)
